# baseline (speedup 1.0000x reference)
import functools

import jax
import jax.numpy as jnp
from jax import lax
from jax.experimental import pallas as pl
from jax.experimental.pallas import tpu as pltpu
from jax.experimental.pallas import tpu_sc as plsc

NC = 2
NS = 16
L = 16

SAMPLE_FRAC = 0.25


def _make_fps(B, N, D, n_samp, interpret=False):
  mesh = plsc.VectorSubcoreMesh(
      core_axis_name="c", subcore_axis_name="s",
      num_cores=NC, num_subcores=NS)

  @functools.partial(
      pl.kernel,
      out_type=(
          jax.ShapeDtypeStruct((B * n_samp,), jnp.float32),
          jax.ShapeDtypeStruct((B * n_samp,), jnp.float32),
          jax.ShapeDtypeStruct((B * n_samp,), jnp.float32),
          jax.ShapeDtypeStruct((B * n_samp, D), jnp.float32),
          jax.ShapeDtypeStruct((B * n_samp,), jnp.float32),
      ),
      mesh=mesh,
      scratch_types=[
          pltpu.VMEM((N,), jnp.float32),
          pltpu.VMEM((N,), jnp.float32),
          pltpu.VMEM((N,), jnp.float32),
          pltpu.VMEM((N,), jnp.float32),
          pltpu.VMEM((N,), jnp.float32),
          pltpu.VMEM((L,), jnp.int32),
          pltpu.VMEM((n_samp,), jnp.int32),
          pltpu.VMEM((n_samp,), jnp.float32),
          pltpu.VMEM((n_samp,), jnp.float32),
          pltpu.VMEM((n_samp,), jnp.float32),
          pltpu.VMEM((n_samp,), jnp.float32),
          pltpu.VMEM((n_samp // 128, 128), jnp.int32),
          pltpu.VMEM((128, D), jnp.float32),
          pltpu.SemaphoreType.DMA,
      ],
      compiler_params=pltpu.CompilerParams(needs_layout_passes=False),
      interpret=interpret,
  )
  def fps(xs, ys, zs, f0_hbm, mask_flat, values_flat,
          qxo, qyo, qzo, outv, outm,
          x_v, y_v, z_v, dist_v, m_v, f0_v, idx_v,
          qx_v, qy_v, qz_v, qm_v, idxg_v, rows_v, sem):
    wid = lax.axis_index("s") * NC + lax.axis_index("c")

    @pl.when(wid < B)
    def _():
      b = wid
      pltpu.sync_copy(xs.at[pl.ds(b * N, N)], x_v)
      pltpu.sync_copy(ys.at[pl.ds(b * N, N)], y_v)
      pltpu.sync_copy(zs.at[pl.ds(b * N, N)], z_v)
      pltpu.sync_copy(mask_flat.at[pl.ds(b * N, N)], m_v)
      pltpu.sync_copy(f0_hbm, f0_v)

      def init_chunk(j, _):
        dist_v[pl.ds(j * L, L)] = jnp.full((L,), 1e8, jnp.float32)
        return 0
      lax.fori_loop(0, N // L, init_chunk, 0)

      lanes = lax.iota(jnp.int32, L)
      bvec = jnp.full((L,), b, jnp.int32)
      fvec0 = plsc.load_gather(f0_v, [bvec])

      big_i = jnp.full((L,), jnp.iinfo(jnp.int32).max, jnp.int32)

      def step(i, carry):
        fvec, idxvec = carry
        t = i % L
        idxvec = jnp.where(lanes == t, fvec, idxvec)
        cx = plsc.load_gather(x_v, [fvec])
        cy = plsc.load_gather(y_v, [fvec])
        cz = plsc.load_gather(z_v, [fvec])

        def chunk(j, c2):
          bv, bi = c2
          sl = pl.ds(j * L, L)
          dx = x_v[sl] - cx
          dy = y_v[sl] - cy
          dz = z_v[sl] - cz
          d = (dz * dz + dx * dx) + dy * dy
          dn = jnp.minimum(d, dist_v[sl])
          dist_v[sl] = dn
          cand = lanes + j * L
          better = (dn > bv) | ((dn == bv) & (cand < bi))
          bv = jnp.where(better, dn, bv)
          bi = jnp.where(better, cand, bi)
          return (bv, bi)

        bv, bi = plsc.parallel_loop(
            0, N // L, step=1, unroll=8,
            carry=(jnp.full((L,), -1.0, jnp.float32),
                   big_i))(chunk)

        m = jnp.max(bv)
        cid = jnp.where(bv == m, bi, big_i)
        fnew = jnp.min(cid)

        @pl.when(t == L - 1)
        def _():
          idx_v[pl.ds((i // L) * L, L)] = idxvec

        return (jnp.full((L,), fnew, jnp.int32), idxvec)

      lax.fori_loop(0, n_samp, step,
                    (fvec0, jnp.zeros((L,), jnp.int32)))

      for g in range(n_samp // L):
        sl = pl.ds(g * L, L)
        iv = idx_v[sl]
        qx_v[sl] = plsc.load_gather(x_v, [iv])
        qy_v[sl] = plsc.load_gather(y_v, [iv])
        qz_v[sl] = plsc.load_gather(z_v, [iv])
        qm_v[sl] = plsc.load_gather(m_v, [iv])
        r, cpos = (g * L) // 128, (g * L) % 128
        idxg_v[r, pl.ds(cpos, L)] = iv + b * N

      pltpu.sync_copy(qx_v, qxo.at[pl.ds(b * n_samp, n_samp)])
      pltpu.sync_copy(qy_v, qyo.at[pl.ds(b * n_samp, n_samp)])
      pltpu.sync_copy(qz_v, qzo.at[pl.ds(b * n_samp, n_samp)])
      pltpu.sync_copy(qm_v, outm.at[pl.ds(b * n_samp, n_samp)])

      for cc in range(n_samp // 128):
        pltpu.async_copy(values_flat.at[idxg_v.at[cc]], rows_v, sem).wait()
        pltpu.sync_copy(rows_v, outv.at[pl.ds(b * n_samp + cc * 128, 128)])

  return fps


@jax.jit
def kernel(coords, values, mask):
  B, N, C = coords.shape
  D = values.shape[-1]
  n_samp = int(round(N * SAMPLE_FRAC))
  f0 = jax.random.randint(jax.random.key(42), (B,), 0, N).astype(jnp.int32)
  xs = coords[:, :, 0].reshape(-1)
  ys = coords[:, :, 1].reshape(-1)
  zs = coords[:, :, 2].reshape(-1)
  values_flat = values.reshape(B * N, D)
  mask_flat = mask.reshape(-1)
  fps = _make_fps(B, N, D, n_samp)
  qx, qy, qz, outv, outm = fps(xs, ys, zs, f0, mask_flat, values_flat)
  query_coords = jnp.stack([qx, qy, qz], axis=-1).reshape(B, n_samp, C)
  return (query_coords, outv.reshape(B, n_samp, D), outm.reshape(B, n_samp))


# speedup vs baseline: 1.3561x; 1.3561x over previous
import functools

import jax
import jax.numpy as jnp
from jax import lax
from jax.experimental import pallas as pl
from jax.experimental.pallas import tpu as pltpu
from jax.experimental.pallas import tpu_sc as plsc

NC = 2
NS = 16
L = 16

SAMPLE_FRAC = 0.25


def _make_fps(B, N, D, n_samp, interpret=False):
  assert B == NC * (NS // 2)
  half_n = N // 2
  mesh = plsc.VectorSubcoreMesh(
      core_axis_name="c", subcore_axis_name="s",
      num_cores=NC, num_subcores=NS)

  @functools.partial(
      pl.kernel,
      out_type=(
          jax.ShapeDtypeStruct((B * n_samp,), jnp.float32),
          jax.ShapeDtypeStruct((B * n_samp,), jnp.float32),
          jax.ShapeDtypeStruct((B * n_samp,), jnp.float32),
          jax.ShapeDtypeStruct((B * n_samp, D), jnp.float32),
          jax.ShapeDtypeStruct((B * n_samp,), jnp.float32),
      ),
      mesh=mesh,
      scratch_types=[
          pltpu.VMEM((N,), jnp.float32),
          pltpu.VMEM((N,), jnp.float32),
          pltpu.VMEM((N,), jnp.float32),
          pltpu.VMEM((N // 2,), jnp.float32),
          pltpu.VMEM((N,), jnp.float32),
          pltpu.VMEM((L,), jnp.int32),
          pltpu.VMEM((n_samp,), jnp.int32),
          pltpu.VMEM((n_samp,), jnp.float32),
          pltpu.VMEM((n_samp,), jnp.float32),
          pltpu.VMEM((n_samp,), jnp.float32),
          pltpu.VMEM((n_samp,), jnp.float32),
          pltpu.VMEM((n_samp // 128, 128), jnp.int32),
          pltpu.VMEM((128, D), jnp.float32),
          pltpu.VMEM((2 * L,), jnp.int32),
          pltpu.VMEM((2 * L,), jnp.int32),
          pltpu.VMEM_SHARED((2 * NS * 2 * L,), jnp.int32),
          pltpu.SemaphoreType.DMA,
      ],
      compiler_params=pltpu.CompilerParams(needs_layout_passes=False),
      interpret=interpret,
  )
  def fps(xs, ys, zs, f0_hbm, mask_flat, values_flat,
          qxo, qyo, qzo, outv, outm,
          x_v, y_v, z_v, dist_v, m_v, f0_v, idx_v,
          qx_v, qy_v, qz_v, qm_v, idxg_v, rows_v, ex_v, px_v, shared, sem):
    s = lax.axis_index("s")
    c = lax.axis_index("c")
    b = c * (NS // 2) + lax.rem(s, NS // 2)
    half = lax.div(s, NS // 2)
    base = half * half_n
    sp = lax.rem(s + NS // 2, NS)

    pltpu.sync_copy(xs.at[pl.ds(b * N, N)], x_v)
    pltpu.sync_copy(ys.at[pl.ds(b * N, N)], y_v)
    pltpu.sync_copy(zs.at[pl.ds(b * N, N)], z_v)
    pltpu.sync_copy(f0_hbm, f0_v)

    @pl.when(half == 0)
    def _():
      pltpu.sync_copy(mask_flat.at[pl.ds(b * N, N)], m_v)

    def init_chunk(j, _):
      dist_v[pl.ds(j * L, L)] = jnp.full((L,), 1e8, jnp.float32)
      return 0
    lax.fori_loop(0, half_n // L, init_chunk, 0)

    lanes = lax.iota(jnp.int32, L)
    bvec = jnp.full((L,), b, jnp.int32)
    fvec0 = plsc.load_gather(f0_v, [bvec])

    big_i = jnp.full((L,), jnp.iinfo(jnp.int32).max, jnp.int32)
    cand0 = lanes + base

    def step(i, carry):
      fvec, idxvec = carry
      t = i % L
      idxvec = jnp.where(lanes == t, fvec, idxvec)
      cx = plsc.load_gather(x_v, [fvec])
      cy = plsc.load_gather(y_v, [fvec])
      cz = plsc.load_gather(z_v, [fvec])

      def chunk(j, c2):
        bv, bi = c2
        sl = pl.ds(j * L, L)
        gsl = pl.ds(base + j * L, L)
        dx = x_v[gsl] - cx
        dy = y_v[gsl] - cy
        dz = z_v[gsl] - cz
        d = (dz * dz + dx * dx) + dy * dy
        dn = jnp.minimum(d, dist_v[sl])
        dist_v[sl] = dn
        cand = cand0 + j * L
        better = (dn > bv) | ((dn == bv) & (cand < bi))
        bv = jnp.where(better, dn, bv)
        bi = jnp.where(better, cand, bi)
        return (bv, bi)

      bv, bi = plsc.parallel_loop(
          0, half_n // L, step=1, unroll=8,
          carry=(jnp.full((L,), -1.0, jnp.float32),
                 big_i))(chunk)

      slot = lax.rem(i, 2)
      ex_v[pl.ds(0, L)] = plsc.bitcast(bv, jnp.int32)
      ex_v[pl.ds(L, L)] = bi
      pltpu.sync_copy(
          ex_v, shared.at[pl.ds(slot * (NS * 2 * L) + s * (2 * L), 2 * L)])
      plsc.subcore_barrier()
      pltpu.sync_copy(
          shared.at[pl.ds(slot * (NS * 2 * L) + sp * (2 * L), 2 * L)], px_v)
      bvp = plsc.bitcast(px_v[pl.ds(0, L)], jnp.float32)
      bip = px_v[pl.ds(L, L)]
      take = (bvp > bv) | ((bvp == bv) & (bip < bi))
      bv = jnp.where(take, bvp, bv)
      bi = jnp.where(take, bip, bi)

      m = jnp.max(bv)
      cid = jnp.where(bv == m, bi, big_i)
      fnew = jnp.min(cid)

      @pl.when(t == L - 1)
      def _():
        idx_v[pl.ds((i // L) * L, L)] = idxvec

      return (jnp.full((L,), fnew, jnp.int32), idxvec)

    lax.fori_loop(0, n_samp, step,
                  (fvec0, jnp.zeros((L,), jnp.int32)))

    @pl.when(half == 0)
    def _():
      for g in range(n_samp // L):
        sl = pl.ds(g * L, L)
        iv = idx_v[sl]
        qx_v[sl] = plsc.load_gather(x_v, [iv])
        qy_v[sl] = plsc.load_gather(y_v, [iv])
        qz_v[sl] = plsc.load_gather(z_v, [iv])
        qm_v[sl] = plsc.load_gather(m_v, [iv])

      pltpu.sync_copy(qx_v, qxo.at[pl.ds(b * n_samp, n_samp)])
      pltpu.sync_copy(qy_v, qyo.at[pl.ds(b * n_samp, n_samp)])
      pltpu.sync_copy(qz_v, qzo.at[pl.ds(b * n_samp, n_samp)])
      pltpu.sync_copy(qm_v, outm.at[pl.ds(b * n_samp, n_samp)])

    n_cc = n_samp // 128
    half_g = n_samp // L // 2
    for gg in range(half_g):
      g = gg + half * half_g
      sl = pl.ds(g * L, L)
      iv = idx_v[sl]
      r = g // 8
      cpos = lax.rem(g, 8) * L
      idxg_v[r, pl.ds(cpos, L)] = iv + b * N

    cc_lo = half * (n_cc // 2)
    for k in range(n_cc // 2):
      cc = cc_lo + k
      pltpu.async_copy(values_flat.at[idxg_v.at[cc]], rows_v, sem).wait()
      pltpu.sync_copy(rows_v, outv.at[pl.ds((b * n_samp + cc * 128) * 1, 128)])

  return fps


@jax.jit
def kernel(coords, values, mask):
  B, N, C = coords.shape
  D = values.shape[-1]
  n_samp = int(round(N * SAMPLE_FRAC))
  f0 = jax.random.randint(jax.random.key(42), (B,), 0, N).astype(jnp.int32)
  xs = coords[:, :, 0].reshape(-1)
  ys = coords[:, :, 1].reshape(-1)
  zs = coords[:, :, 2].reshape(-1)
  values_flat = values.reshape(B * N, D)
  mask_flat = mask.reshape(-1)
  fps = _make_fps(B, N, D, n_samp)
  qx, qy, qz, outv, outm = fps(xs, ys, zs, f0, mask_flat, values_flat)
  query_coords = jnp.stack([qx, qy, qz], axis=-1).reshape(B, n_samp, C)
  return (query_coords, outv.reshape(B, n_samp, D), outm.reshape(B, n_samp))


# speedup vs baseline: 1.3897x; 1.0248x over previous
import functools

import jax
import jax.numpy as jnp
from jax import lax
from jax.experimental import pallas as pl
from jax.experimental.pallas import tpu as pltpu
from jax.experimental.pallas import tpu_sc as plsc

NC = 2
NS = 16
L = 16

SAMPLE_FRAC = 0.25


def _make_fps(B, N, D, n_samp, interpret=False):
  assert B == NC * (NS // 2)
  half_n = N // 2
  mesh = plsc.VectorSubcoreMesh(
      core_axis_name="c", subcore_axis_name="s",
      num_cores=NC, num_subcores=NS)

  @functools.partial(
      pl.kernel,
      out_type=(
          jax.ShapeDtypeStruct((B * n_samp,), jnp.float32),
          jax.ShapeDtypeStruct((B * n_samp,), jnp.float32),
          jax.ShapeDtypeStruct((B * n_samp,), jnp.float32),
          jax.ShapeDtypeStruct((B * n_samp, D), jnp.float32),
          jax.ShapeDtypeStruct((B * n_samp,), jnp.float32),
      ),
      mesh=mesh,
      scratch_types=[
          pltpu.VMEM((N,), jnp.float32),
          pltpu.VMEM((N,), jnp.float32),
          pltpu.VMEM((N,), jnp.float32),
          pltpu.VMEM((N // 2,), jnp.float32),
          pltpu.VMEM((N,), jnp.float32),
          pltpu.VMEM((L,), jnp.int32),
          pltpu.VMEM((n_samp,), jnp.int32),
          pltpu.VMEM((n_samp,), jnp.float32),
          pltpu.VMEM((n_samp,), jnp.float32),
          pltpu.VMEM((n_samp,), jnp.float32),
          pltpu.VMEM((n_samp,), jnp.float32),
          pltpu.VMEM((n_samp // 128, 128), jnp.int32),
          pltpu.VMEM((128, D), jnp.float32),
          pltpu.VMEM((2 * L,), jnp.int32),
          pltpu.VMEM((2 * L,), jnp.int32),
          pltpu.VMEM_SHARED((2 * NS * 2 * L,), jnp.int32),
          pltpu.SemaphoreType.DMA,
      ],
      compiler_params=pltpu.CompilerParams(needs_layout_passes=False),
      interpret=interpret,
  )
  def fps(xs, ys, zs, f0_hbm, mask_flat, values_flat,
          qxo, qyo, qzo, outv, outm,
          x_v, y_v, z_v, dist_v, m_v, f0_v, idx_v,
          qx_v, qy_v, qz_v, qm_v, idxg_v, rows_v, ex_v, px_v, shared, sem):
    s = lax.axis_index("s")
    c = lax.axis_index("c")
    b = c * (NS // 2) + lax.rem(s, NS // 2)
    half = lax.div(s, NS // 2)
    base = half * half_n
    sp = lax.rem(s + NS // 2, NS)

    pltpu.sync_copy(xs.at[pl.ds(b * N, N)], x_v)
    pltpu.sync_copy(ys.at[pl.ds(b * N, N)], y_v)
    pltpu.sync_copy(zs.at[pl.ds(b * N, N)], z_v)
    pltpu.sync_copy(f0_hbm, f0_v)

    @pl.when(half == 0)
    def _():
      pltpu.sync_copy(mask_flat.at[pl.ds(b * N, N)], m_v)

    def init_chunk(j, _):
      dist_v[pl.ds(j * L, L)] = jnp.full((L,), 1e8, jnp.float32)
      return 0
    lax.fori_loop(0, half_n // L, init_chunk, 0)

    lanes = lax.iota(jnp.int32, L)
    bvec = jnp.full((L,), b, jnp.int32)
    fvec0 = plsc.load_gather(f0_v, [bvec])

    big_i = jnp.full((L,), jnp.iinfo(jnp.int32).max, jnp.int32)
    cand0 = lanes + base

    def step(i, carry):
      fvec, idxvec = carry
      t = i % L
      idxvec = jnp.where(lanes == t, fvec, idxvec)
      cx = plsc.load_gather(x_v, [fvec])
      cy = plsc.load_gather(y_v, [fvec])
      cz = plsc.load_gather(z_v, [fvec])

      def chunk(j, c2):
        bv, bi = c2
        sl = pl.ds(j * L, L)
        gsl = pl.ds(base + j * L, L)
        dx = x_v[gsl] - cx
        dy = y_v[gsl] - cy
        dz = z_v[gsl] - cz
        d = (dz * dz + dx * dx) + dy * dy
        dn = jnp.minimum(d, dist_v[sl])
        dist_v[sl] = dn
        cand = cand0 + j * L
        better = (dn > bv) | ((dn == bv) & (cand < bi))
        bv = jnp.where(better, dn, bv)
        bi = jnp.where(better, cand, bi)
        return (bv, bi)

      bv, bi = plsc.parallel_loop(
          0, half_n // L, step=1, unroll=4,
          carry=(jnp.full((L,), -1.0, jnp.float32),
                 big_i))(chunk)

      slot = lax.rem(i, 2)
      ex_v[pl.ds(0, L)] = plsc.bitcast(bv, jnp.int32)
      ex_v[pl.ds(L, L)] = bi
      pltpu.sync_copy(
          ex_v, shared.at[pl.ds(slot * (NS * 2 * L) + s * (2 * L), 2 * L)])
      plsc.subcore_barrier()
      pltpu.sync_copy(
          shared.at[pl.ds(slot * (NS * 2 * L) + sp * (2 * L), 2 * L)], px_v)
      bvp = plsc.bitcast(px_v[pl.ds(0, L)], jnp.float32)
      bip = px_v[pl.ds(L, L)]
      take = (bvp > bv) | ((bvp == bv) & (bip < bi))
      bv = jnp.where(take, bvp, bv)
      bi = jnp.where(take, bip, bi)

      m = jnp.max(bv)
      cid = jnp.where(bv == m, bi, big_i)
      fnew = jnp.min(cid)

      @pl.when(t == L - 1)
      def _():
        idx_v[pl.ds((i // L) * L, L)] = idxvec

      return (jnp.full((L,), fnew, jnp.int32), idxvec)

    lax.fori_loop(0, n_samp, step,
                  (fvec0, jnp.zeros((L,), jnp.int32)))

    @pl.when(half == 0)
    def _():
      for g in range(n_samp // L):
        sl = pl.ds(g * L, L)
        iv = idx_v[sl]
        qx_v[sl] = plsc.load_gather(x_v, [iv])
        qy_v[sl] = plsc.load_gather(y_v, [iv])
        qz_v[sl] = plsc.load_gather(z_v, [iv])
        qm_v[sl] = plsc.load_gather(m_v, [iv])

      pltpu.sync_copy(qx_v, qxo.at[pl.ds(b * n_samp, n_samp)])
      pltpu.sync_copy(qy_v, qyo.at[pl.ds(b * n_samp, n_samp)])
      pltpu.sync_copy(qz_v, qzo.at[pl.ds(b * n_samp, n_samp)])
      pltpu.sync_copy(qm_v, outm.at[pl.ds(b * n_samp, n_samp)])

    n_cc = n_samp // 128
    half_g = n_samp // L // 2
    for gg in range(half_g):
      g = gg + half * half_g
      sl = pl.ds(g * L, L)
      iv = idx_v[sl]
      r = g // 8
      cpos = lax.rem(g, 8) * L
      idxg_v[r, pl.ds(cpos, L)] = iv + b * N

    cc_lo = half * (n_cc // 2)
    for k in range(n_cc // 2):
      cc = cc_lo + k
      pltpu.async_copy(values_flat.at[idxg_v.at[cc]], rows_v, sem).wait()
      pltpu.sync_copy(rows_v, outv.at[pl.ds((b * n_samp + cc * 128) * 1, 128)])

  return fps


@jax.jit
def kernel(coords, values, mask):
  B, N, C = coords.shape
  D = values.shape[-1]
  n_samp = int(round(N * SAMPLE_FRAC))
  f0 = jax.random.randint(jax.random.key(42), (B,), 0, N).astype(jnp.int32)
  xs = coords[:, :, 0].reshape(-1)
  ys = coords[:, :, 1].reshape(-1)
  zs = coords[:, :, 2].reshape(-1)
  values_flat = values.reshape(B * N, D)
  mask_flat = mask.reshape(-1)
  fps = _make_fps(B, N, D, n_samp)
  qx, qy, qz, outv, outm = fps(xs, ys, zs, f0, mask_flat, values_flat)
  query_coords = jnp.stack([qx, qy, qz], axis=-1).reshape(B, n_samp, C)
  return (query_coords, outv.reshape(B, n_samp, D), outm.reshape(B, n_samp))
